# Initial kernel scaffold; baseline (speedup 1.0000x reference)
import jax
import jax.numpy as jnp
from jax.experimental import pallas as pl


def kernel(x, edge_index, params):
    raise NotImplementedError("write your pallas kernel here")


# speedup vs baseline: 5.0700x; 5.0700x over previous
import functools

import jax
import jax.numpy as jnp
from jax import lax
from jax.experimental import pallas as pl
from jax.experimental.pallas import tpu as pltpu
from jax.experimental.pallas import tpu_sc as plsc

NC = 2
NS = 16
NW = NC * NS
CHUNK = 128


def _make_seg_sum(n_pad, dh, e_pad):
    ch_total = e_pad // CHUNK
    ch_per_tile = ch_total // NS
    t_steps = ch_per_tile // 2
    rpt = n_pad // NS
    mesh = plsc.VectorSubcoreMesh(core_axis_name="c", subcore_axis_name="s")

    @functools.partial(
        pl.kernel,
        out_type=jax.ShapeDtypeStruct((NC, n_pad, dh), jnp.float32),
        mesh=mesh,
        scratch_types=[
            pltpu.VMEM((ch_per_tile, CHUNK), jnp.int32),
            pltpu.VMEM((ch_per_tile, CHUNK), jnp.int32),
            pltpu.VMEM((CHUNK, dh), jnp.float32),
            pltpu.VMEM((CHUNK, dh), jnp.float32),
            pltpu.VMEM_SHARED((n_pad, dh), jnp.float32),
            pltpu.SemaphoreType.DMA,
            pltpu.SemaphoreType.DMA,
        ],
        compiler_params=pltpu.CompilerParams(use_tc_tiling_on_sc=False),
    )
    def seg_sum(x_hbm, src_hbm, dst_hbm, zeros_hbm, out_hbm,
                src_v, dst_v, buf0, buf1, agg_s, sem0, sem1):
        cid = lax.axis_index("c")
        sid = lax.axis_index("s")
        base = sid * ch_per_tile
        pltpu.sync_copy(src_hbm.at[cid].at[pl.ds(base, ch_per_tile)], src_v)
        pltpu.sync_copy(dst_hbm.at[pl.ds(base, ch_per_tile)], dst_v)
        row0 = sid * rpt
        pltpu.sync_copy(zeros_hbm.at[pl.ds(row0, rpt)], agg_s.at[pl.ds(row0, rpt)])
        plsc.subcore_barrier()

        def start(j, buf, sem):
            pltpu.async_copy(x_hbm.at[src_v.at[j]], buf, sem)

        def finish(j, buf, sem):
            pltpu.make_async_copy(x_hbm.at[src_v.at[j]], buf, sem).wait()
            pltpu.sync_copy(buf, agg_s.at[dst_v.at[j]], add=True)

        start(0, buf0, sem0)

        def body(t, carry):
            start(2 * t + 1, buf1, sem1)
            finish(2 * t, buf0, sem0)

            @pl.when(t < t_steps - 1)
            def _():
                start(2 * t + 2, buf0, sem0)

            finish(2 * t + 1, buf1, sem1)
            return carry

        lax.fori_loop(0, t_steps, body, 0)
        plsc.subcore_barrier()
        pltpu.sync_copy(agg_s.at[pl.ds(row0, rpt)],
                        out_hbm.at[cid].at[pl.ds(row0, rpt)])

    return seg_sum


def _hidden_layer_body(n):
    def body(x_ref, part_ref, w1_ref, b1_ref, w2_ref, b2_ref, g_ref, be_ref,
             eps_ref, out_ref):
        agg = jnp.concatenate([part_ref[0, :n, :], part_ref[1, :n, :]], axis=1)
        h = x_ref[...] * (1.0 + eps_ref[0]) + agg
        h = jnp.maximum(jnp.dot(h, w1_ref[...], preferred_element_type=jnp.float32)
                        + b1_ref[...], 0.0)
        h = jnp.maximum(jnp.dot(h, w2_ref[...], preferred_element_type=jnp.float32)
                        + b2_ref[...], 0.0)
        mean = jnp.mean(h, axis=0, keepdims=True)
        var = jnp.mean((h - mean) ** 2, axis=0, keepdims=True)
        h = (h - mean) / jnp.sqrt(var + 1e-5) * g_ref[...] + be_ref[...]
        out_ref[...] = jnp.maximum(h, 0.0)
    return body


def _final_layer_body(n):
    def body(x_ref, part_ref, w1_ref, b1_ref, w2_ref, b2_ref, g_ref, be_ref,
             eps_ref, lw_ref, lb_ref, out_ref):
        agg = jnp.concatenate([part_ref[0, :n, :], part_ref[1, :n, :]], axis=1)
        h = x_ref[...] * (1.0 + eps_ref[0]) + agg
        h = jnp.maximum(jnp.dot(h, w1_ref[...], preferred_element_type=jnp.float32)
                        + b1_ref[...], 0.0)
        h = jnp.maximum(jnp.dot(h, w2_ref[...], preferred_element_type=jnp.float32)
                        + b2_ref[...], 0.0)
        mean = jnp.mean(h, axis=0, keepdims=True)
        var = jnp.mean((h - mean) ** 2, axis=0, keepdims=True)
        h = (h - mean) / jnp.sqrt(var + 1e-5) * g_ref[...] + be_ref[...]
        h = jnp.maximum(h, 0.0)
        out_ref[...] = (jnp.dot(h, lw_ref[...], preferred_element_type=jnp.float32)
                        + lb_ref[...])
    return body


def _vmem():
    return pl.BlockSpec(memory_space=pltpu.VMEM)


def _smem():
    return pl.BlockSpec(memory_space=pltpu.SMEM)


def kernel(x, edge_index, params):
    n, _ = x.shape
    e = edge_index.shape[1]
    n_pad = -(-(n + 1) // (NS * 8)) * (NS * 8)
    block = NS * CHUNK * 8
    e_pad = -(-e // block) * block
    pad = e_pad - e
    src = edge_index[0].astype(jnp.int32)
    dst = edge_index[1].astype(jnp.int32)
    src_p = jnp.concatenate([src, jnp.zeros((pad,), jnp.int32)])
    src_p = jnp.stack([2 * src_p, 2 * src_p + 1]).reshape(2, -1, CHUNK)
    dst_p = jnp.concatenate([dst, jnp.full((pad,), n, jnp.int32)]).reshape(-1, CHUNK)

    h = x
    for i in range(3):
        d = h.shape[1]
        dh = d // 2
        hdim = params[f"conv{i+1}_w1"].shape[1]
        zeros = jnp.zeros((n_pad, dh), jnp.float32)
        part = _make_seg_sum(n_pad, dh, e_pad)(
            h.reshape(2 * n, dh), src_p, dst_p, zeros)
        eps1 = params[f"conv{i+1}_eps"].reshape(1)
        args = [h, part,
                params[f"conv{i+1}_w1"], params[f"conv{i+1}_b1"].reshape(1, hdim),
                params[f"conv{i+1}_w2"], params[f"conv{i+1}_b2"].reshape(1, hdim),
                params[f"conv{i+1}_gamma"].reshape(1, hdim),
                params[f"conv{i+1}_beta"].reshape(1, hdim), eps1]
        last = i == 2
        if last:
            c = params["lin_w"].shape[1]
            args += [params["lin_w"], params["lin_b"].reshape(1, c)]
            body = _final_layer_body(n)
            out_shape = jax.ShapeDtypeStruct((n, c), jnp.float32)
        else:
            body = _hidden_layer_body(n)
            out_shape = jax.ShapeDtypeStruct((n, hdim), jnp.float32)
        in_specs = [_vmem()] * len(args)
        in_specs[8] = _smem()
        h = pl.pallas_call(
            body,
            out_shape=out_shape,
            in_specs=in_specs,
            out_specs=_vmem(),
        )(*args)
    return h
